# initial kernel scaffold (unmeasured)
import jax
import jax.numpy as jnp
from jax import lax
from jax.experimental import pallas as pl
from jax.experimental.pallas import tpu as pltpu


def kernel(
    x,
):
    def body(*refs):
        pass

    out_shape = jax.ShapeDtypeStruct(..., jnp.float32)
    return pl.pallas_call(body, out_shape=out_shape)(...)


# baseline (device time: 20872 ns/iter reference)
import functools

import jax
import jax.numpy as jnp
from jax import lax
from jax.experimental import pallas as pl
from jax.experimental.pallas import tpu as pltpu

N_DEV = 32

_sem_signal = getattr(pl, "semaphore_signal", None) or pltpu.semaphore_signal
_sem_wait = getattr(pl, "semaphore_wait", None) or pltpu.semaphore_wait
_CompilerParams = getattr(pltpu, "CompilerParams", None) or pltpu.TPUCompilerParams
_DeviceIdType = getattr(pl, "DeviceIdType", None) or pltpu.DeviceIdType


def kernel(x):
    m_per, n = x.shape

    def body(x_ref, out_ref, comm_ref, send_sems, recv_sems):
        my_pos = lax.axis_index("i")

        barrier_sem = pltpu.get_barrier_semaphore()
        for d in range(1, N_DEV):
            _sem_signal(
                barrier_sem,
                inc=1,
                device_id=((my_pos + d) % N_DEV,),
                device_id_type=_DeviceIdType.MESH,
            )
        _sem_wait(barrier_sem, N_DEV - 1)

        comm_ref[pl.ds(my_pos, 1), :] = jnp.max(x_ref[:, :], axis=0, keepdims=True)

        sends = []
        for d in range(1, N_DEV):
            s = pltpu.make_async_remote_copy(
                src_ref=comm_ref.at[my_pos],
                dst_ref=comm_ref.at[my_pos],
                send_sem=send_sems.at[d],
                recv_sem=recv_sems.at[my_pos],
                device_id=((my_pos + d) % N_DEV,),
                device_id_type=_DeviceIdType.MESH,
            )
            s.start()
            sends.append(s)

        for d in range(1, N_DEV):
            src_pos = (my_pos + d) % N_DEV
            recv = pltpu.make_async_remote_copy(
                src_ref=comm_ref.at[src_pos],
                dst_ref=comm_ref.at[src_pos],
                send_sem=send_sems.at[d],
                recv_sem=recv_sems.at[src_pos],
                device_id=(my_pos,),
                device_id_type=_DeviceIdType.MESH,
            )
            recv.wait_recv()

        out_ref[:, :] = jnp.max(comm_ref[:, :], axis=0, keepdims=True)

        for s in sends:
            s.wait_send()

        @functools.partial(
            pl.run_scoped, second_barrier=pltpu.SemaphoreType.REGULAR
        )
        def _(second_barrier):
            for d in range(1, N_DEV):
                _sem_signal(
                    second_barrier,
                    inc=1,
                    device_id=((my_pos + d) % N_DEV,),
                    device_id_type=_DeviceIdType.MESH,
                )
            _sem_wait(second_barrier, N_DEV - 1)

    return pl.pallas_call(
        body,
        out_shape=jax.ShapeDtypeStruct((1, n), x.dtype),
        in_specs=[pl.BlockSpec(memory_space=pltpu.VMEM)],
        out_specs=pl.BlockSpec(memory_space=pltpu.VMEM),
        scratch_shapes=[
            pltpu.VMEM((N_DEV, n), x.dtype),
            pltpu.SemaphoreType.DMA((N_DEV,)),
            pltpu.SemaphoreType.DMA((N_DEV,)),
        ],
        compiler_params=_CompilerParams(collective_id=0),
    )(x)
